# Initial kernel scaffold; baseline (speedup 1.0000x reference)
import jax
import jax.numpy as jnp
from jax.experimental import pallas as pl


def kernel(center_table, context_table, center_word, context_word, negative_samples):
    raise NotImplementedError("write your pallas kernel here")


# speedup vs baseline: 2.6861x; 2.6861x over previous
import functools

import jax
import jax.numpy as jnp
from jax import lax
from jax.experimental import pallas as pl
from jax.experimental.pallas import tpu as pltpu
from jax.experimental.pallas import tpu_sc as plsc

_VOCAB = 100000
_DIM = 64
_B = 16384
_NEG = 20

_NC = 2
_NS = 16
_NW = _NC * _NS
_CH = 512


def _sc_gather(center_table, context_table, cidx, xidx, nidx):
    mesh = plsc.VectorSubcoreMesh(core_axis_name="c", subcore_axis_name="s")
    n_total = nidx.shape[0]
    c_per_w = _B // _NW
    n_per_w = n_total // _NW

    @functools.partial(
        pl.kernel,
        mesh=mesh,
        compiler_params=pltpu.CompilerParams(use_tc_tiling_on_sc=False),
        out_type=[
            jax.ShapeDtypeStruct((_B, _DIM), jnp.float32),
            jax.ShapeDtypeStruct((_B, _DIM), jnp.float32),
            jax.ShapeDtypeStruct((n_total, _DIM), jnp.float32),
        ],
        scratch_types=[
            pltpu.VMEM((_CH,), jnp.int32),
            pltpu.VMEM((_CH, _DIM), jnp.float32),
            pltpu.SemaphoreType.DMA,
        ],
    )
    def k(ctab_hbm, xtab_hbm, cidx_hbm, xidx_hbm, nidx_hbm,
          c_out, x_out, n_out, idx_v, rows_v, sem):
        wid = lax.axis_index("s") * _NC + lax.axis_index("c")

        def gather_chunk(tab_hbm, idx_hbm, out_hbm, base):
            pltpu.sync_copy(idx_hbm.at[pl.ds(base, _CH)], idx_v)
            pltpu.async_copy(tab_hbm.at[idx_v], rows_v, sem).wait()
            pltpu.sync_copy(rows_v, out_hbm.at[pl.ds(base, _CH)])

        @pl.loop(0, c_per_w // _CH)
        def _(ci):
            gather_chunk(ctab_hbm, cidx_hbm, c_out, wid * c_per_w + ci * _CH)

        @pl.loop(0, c_per_w // _CH)
        def _(ci):
            gather_chunk(xtab_hbm, xidx_hbm, x_out, wid * c_per_w + ci * _CH)

        @pl.loop(0, n_per_w // _CH)
        def _(ci):
            gather_chunk(xtab_hbm, nidx_hbm, n_out, wid * n_per_w + ci * _CH)

    return k(center_table, context_table, cidx, xidx, nidx)


_BS = 2048


def _tc_body(c_ref, x_ref, n_ref, pos_ref, neg_ref):
    i = pl.program_id(0)
    j = pl.program_id(1)
    eps = 1e-12

    @pl.when(jnp.logical_and(i == 0, j == 0))
    def _():
        pos_ref[...] = jnp.zeros((1, 1), jnp.float32)
        neg_ref[...] = jnp.zeros((1, 1), jnp.float32)

    c = c_ref[...]
    cn = c / jnp.maximum(jnp.sqrt(jnp.sum(c * c, axis=1, keepdims=True)), eps)

    @pl.when(j == 0)
    def _():
        x = x_ref[...]
        xn = x / jnp.maximum(jnp.sqrt(jnp.sum(x * x, axis=1, keepdims=True)), eps)
        pos = jnp.sum(cn * xn, axis=1)
        ps = jnp.clip(jax.nn.sigmoid(pos), 1e-6, 1.0 - 1e-6)
        pos_ref[...] += -jnp.sum(jnp.log(ps)).reshape(1, 1)

    n = n_ref[...]
    nn = n / jnp.maximum(jnp.sqrt(jnp.sum(n * n, axis=1, keepdims=True)), eps)
    s = jnp.sum(nn * cn, axis=1)
    ns = jnp.clip(jax.nn.sigmoid(s), 1e-6, 1.0 - 1e-6)
    neg_ref[...] += -jnp.sum(jnp.log(1.0 - ns)).reshape(1, 1)


def _tc_loss(c_rows, x_rows, n_rows):
    nb = _B // _BS
    pos_s, neg_s = pl.pallas_call(
        _tc_body,
        grid=(nb, _NEG),
        in_specs=[
            pl.BlockSpec((_BS, _DIM), lambda i, j: (i, 0)),
            pl.BlockSpec((_BS, _DIM), lambda i, j: (i, 0)),
            pl.BlockSpec((_BS, _DIM), lambda i, j: (j * nb + i, 0)),
        ],
        out_specs=[
            pl.BlockSpec((1, 1), lambda i, j: (0, 0)),
            pl.BlockSpec((1, 1), lambda i, j: (0, 0)),
        ],
        out_shape=[
            jax.ShapeDtypeStruct((1, 1), jnp.float32),
            jax.ShapeDtypeStruct((1, 1), jnp.float32),
        ],
    )(c_rows, x_rows, n_rows)
    return pos_s[0, 0], neg_s[0, 0]


def kernel(center_table, context_table, center_word, context_word, negative_samples):
    cidx = jnp.clip(center_word, 0, _VOCAB - 1).astype(jnp.int32)
    xidx = jnp.clip(context_word, 0, _VOCAB - 1).astype(jnp.int32)
    nidx = jnp.clip(negative_samples, 0, _VOCAB - 1).astype(jnp.int32).T.reshape(-1)

    c_rows, x_rows, n_rows = _sc_gather(center_table, context_table, cidx, xidx, nidx)
    pos_sum, neg_sum = _tc_loss(c_rows, x_rows, n_rows)

    pos_loss = pos_sum / _B
    neg_loss = neg_sum / (_B * _NEG)
    return (pos_loss + neg_loss, pos_loss, neg_loss)


# speedup vs baseline: 3.6515x; 1.3594x over previous
import functools

import jax
import jax.numpy as jnp
from jax import lax
from jax.experimental import pallas as pl
from jax.experimental.pallas import tpu as pltpu
from jax.experimental.pallas import tpu_sc as plsc

_VOCAB = 100000
_DIM = 64
_B = 16384
_NEG = 20

_NC = 2
_NS = 16
_NW = _NC * _NS
_CH = 512


def _sc_gather(center_table, context_table, cidx, xidx, nidx):
    mesh = plsc.VectorSubcoreMesh(core_axis_name="c", subcore_axis_name="s")
    n_total = nidx.shape[0]
    c_per_w = _B // _NW
    n_per_w = n_total // _NW

    @functools.partial(
        pl.kernel,
        mesh=mesh,
        compiler_params=pltpu.CompilerParams(use_tc_tiling_on_sc=False),
        out_type=[
            jax.ShapeDtypeStruct((_B, _DIM), jnp.float32),
            jax.ShapeDtypeStruct((_B, _DIM), jnp.float32),
            jax.ShapeDtypeStruct((n_total, _DIM), jnp.float32),
        ],
        scratch_types=[
            pltpu.VMEM((_CH,), jnp.int32),
            pltpu.VMEM((_CH, _DIM), jnp.float32),
            pltpu.SemaphoreType.DMA,
        ],
    )
    def k(ctab_hbm, xtab_hbm, cidx_hbm, xidx_hbm, nidx_hbm,
          c_out, x_out, n_out, idx_v, rows_v, sem):
        wid = lax.axis_index("s") * _NC + lax.axis_index("c")

        def gather_chunk(tab_hbm, idx_hbm, out_hbm, base):
            pltpu.sync_copy(idx_hbm.at[pl.ds(base, _CH)], idx_v)
            pltpu.async_copy(tab_hbm.at[idx_v], rows_v, sem).wait()
            pltpu.sync_copy(rows_v, out_hbm.at[pl.ds(base, _CH)])

        @pl.loop(0, c_per_w // _CH)
        def _(ci):
            gather_chunk(ctab_hbm, cidx_hbm, c_out, wid * c_per_w + ci * _CH)

        @pl.loop(0, c_per_w // _CH)
        def _(ci):
            gather_chunk(xtab_hbm, xidx_hbm, x_out, wid * c_per_w + ci * _CH)

        @pl.loop(0, n_per_w // _CH)
        def _(ci):
            gather_chunk(xtab_hbm, nidx_hbm, n_out, wid * n_per_w + ci * _CH)

    return k(center_table, context_table, cidx, xidx, nidx)


_BSP = 1024
_EPS = 1e-12


def _half_sums(v):
    return (jnp.sum(v[:, :_DIM], axis=1, keepdims=True),
            jnp.sum(v[:, _DIM:], axis=1, keepdims=True))


def _score_loss(p_l, p_r, ss_l, ss_r, one_minus):
    s_l = p_l / jnp.maximum(jnp.sqrt(ss_l), _EPS)
    s_r = p_r / jnp.maximum(jnp.sqrt(ss_r), _EPS)
    s = jnp.concatenate([s_l, s_r], axis=1)
    sg = jnp.clip(jax.nn.sigmoid(s), 1e-6, 1.0 - 1e-6)
    if one_minus:
        sg = 1.0 - sg
    return -jnp.sum(jnp.log(sg))


def _tc_body(c_ref, x_ref, n_ref, pos_ref, neg_ref, cn_ref):
    j = pl.program_id(1)

    @pl.when(j == 0)
    def _():
        pos_ref[...] = jnp.zeros((1, 1, 1), jnp.float32)
        neg_ref[...] = jnp.zeros((1, 1, 1), jnp.float32)
        c = c_ref[...]
        cc_l, cc_r = _half_sums(c * c)
        inv_l = 1.0 / jnp.maximum(jnp.sqrt(cc_l), _EPS)
        inv_r = 1.0 / jnp.maximum(jnp.sqrt(cc_r), _EPS)
        lane = jax.lax.broadcasted_iota(jnp.int32, (_BSP, 2 * _DIM), 1)
        inv = jnp.where(lane < _DIM, inv_l, inv_r)
        cn = c * inv
        cn_ref[...] = cn
        x = x_ref[...]
        p_l, p_r = _half_sums(x * cn)
        xx_l, xx_r = _half_sums(x * x)
        pos_ref[...] += _score_loss(p_l, p_r, xx_l, xx_r, False).reshape(1, 1, 1)

    n = n_ref[...]
    cn = cn_ref[...]
    p_l, p_r = _half_sums(n * cn)
    nn_l, nn_r = _half_sums(n * n)
    neg_ref[...] += _score_loss(p_l, p_r, nn_l, nn_r, True).reshape(1, 1, 1)


def _tc_loss(c_rows, x_rows, n_rows):
    c128 = c_rows.reshape(_B // 2, 2 * _DIM)
    x128 = x_rows.reshape(_B // 2, 2 * _DIM)
    n128 = n_rows.reshape(_B * _NEG // 2, 2 * _DIM)
    nb = (_B // 2) // _BSP
    pos_s, neg_s = pl.pallas_call(
        _tc_body,
        grid=(nb, _NEG),
        in_specs=[
            pl.BlockSpec((_BSP, 2 * _DIM), lambda i, j: (i, 0)),
            pl.BlockSpec((_BSP, 2 * _DIM), lambda i, j: (i, 0)),
            pl.BlockSpec((_BSP, 2 * _DIM), lambda i, j: (j * nb + i, 0)),
        ],
        out_specs=[
            pl.BlockSpec((1, 1, 1), lambda i, j: (i, 0, 0)),
            pl.BlockSpec((1, 1, 1), lambda i, j: (i, 0, 0)),
        ],
        out_shape=[
            jax.ShapeDtypeStruct((nb, 1, 1), jnp.float32),
            jax.ShapeDtypeStruct((nb, 1, 1), jnp.float32),
        ],
        scratch_shapes=[pltpu.VMEM((_BSP, 2 * _DIM), jnp.float32)],
        compiler_params=pltpu.CompilerParams(
            dimension_semantics=("parallel", "arbitrary")),
    )(c128, x128, n128)
    return jnp.sum(pos_s), jnp.sum(neg_s)


def kernel(center_table, context_table, center_word, context_word, negative_samples):
    cidx = jnp.clip(center_word, 0, _VOCAB - 1).astype(jnp.int32)
    xidx = jnp.clip(context_word, 0, _VOCAB - 1).astype(jnp.int32)
    nidx = jnp.clip(negative_samples, 0, _VOCAB - 1).astype(jnp.int32).T.reshape(-1)

    c_rows, x_rows, n_rows = _sc_gather(center_table, context_table, cidx, xidx, nidx)
    pos_sum, neg_sum = _tc_loss(c_rows, x_rows, n_rows)

    pos_loss = pos_sum / _B
    neg_loss = neg_sum / (_B * _NEG)
    return (pos_loss + neg_loss, pos_loss, neg_loss)
